# baseline (device time: 35261 ns/iter reference)
import jax
import jax.numpy as jnp
from jax import lax
from jax.experimental import pallas as pl
from jax.experimental.pallas import tpu as pltpu

N_DEV = 32


def kernel(x, w_mat):
    m_per, k = x.shape
    _, n_per = w_mat.shape
    m = N_DEV * m_per

    def body(x_ref, w_ref, out_ref, xfull_ref, send_sems, recv_sems):
        me = lax.axis_index("i")

        barrier_sem = pltpu.get_barrier_semaphore()
        for d in range(1, N_DEV):
            pl.semaphore_signal(
                barrier_sem, inc=1,
                device_id=((me + d) % N_DEV,),
                device_id_type=pl.DeviceIdType.MESH,
            )
        pl.semaphore_wait(barrier_sem, N_DEV - 1)

        my_rows = pl.ds(me * m_per, m_per)
        xfull_ref[my_rows, :] = x_ref[:, :].astype(jnp.bfloat16)

        sends = []
        for d in range(1, N_DEV):
            rdma = pltpu.make_async_remote_copy(
                src_ref=xfull_ref.at[my_rows, :],
                dst_ref=xfull_ref.at[my_rows, :],
                send_sem=send_sems.at[d],
                recv_sem=recv_sems.at[d],
                device_id=((me + d) % N_DEV,),
                device_id_type=pl.DeviceIdType.MESH,
            )
            rdma.start()
            sends.append(rdma)

        for d in range(1, N_DEV):
            src_rows = pl.ds(((me - d) % N_DEV) * m_per, m_per)
            recv = pltpu.make_async_remote_copy(
                src_ref=xfull_ref.at[src_rows, :],
                dst_ref=xfull_ref.at[src_rows, :],
                send_sem=send_sems.at[d],
                recv_sem=recv_sems.at[d],
                device_id=((me + d) % N_DEV,),
                device_id_type=pl.DeviceIdType.MESH,
            )
            recv.wait_recv()

        y = lax.dot_general(
            xfull_ref[:, :], w_ref[:, :].astype(jnp.bfloat16),
            (((1,), (0,)), ((), ())),
            preferred_element_type=jnp.float32,
        )
        c = 0.7978845608028654
        out_ref[:, :] = 0.5 * y * (1.0 + jnp.tanh(c * (y + 0.044715 * y * y * y)))

        for rdma in sends:
            rdma.wait_send()

    return pl.pallas_call(
        body,
        out_shape=jax.ShapeDtypeStruct((m, n_per), jnp.float32),
        in_specs=[
            pl.BlockSpec(memory_space=pltpu.VMEM),
            pl.BlockSpec(memory_space=pltpu.VMEM),
        ],
        out_specs=pl.BlockSpec(memory_space=pltpu.VMEM),
        scratch_shapes=[
            pltpu.VMEM((m, k), jnp.bfloat16),
            pltpu.SemaphoreType.DMA((N_DEV,)),
            pltpu.SemaphoreType.DMA((N_DEV,)),
        ],
        compiler_params=pltpu.CompilerParams(collective_id=0),
    )(x, w_mat)


# device time: 26328 ns/iter; 1.3393x vs baseline; 1.3393x over previous
import jax
import jax.numpy as jnp
from jax import lax
from jax.experimental import pallas as pl
from jax.experimental.pallas import tpu as pltpu

N_DEV = 32
NZ = 4
NY = 4

_ZID = lambda dz: dz - 1
_YID = lambda dy, s: 3 + (dy - 1) * 4 + s
_XID = lambda dyo, dzo: 15 + dyo * 4 + dzo
N_SEMS = 31


def kernel(x, w_mat):
    m_per, k = x.shape
    _, n_per = w_mat.shape
    m = N_DEV * m_per

    def body(x_ref, w_ref, out_ref, xfull_ref, send_sems, recv_sems):
        me = lax.axis_index("i")
        my_z = me // 8
        r = me % 8
        my_y = r // 2
        b = r % 2

        def idx_same_x(y, z):
            return 8 * z + 2 * y + jnp.bitwise_xor(b, jnp.bitwise_and(my_y ^ y, 1))

        def chunk_copy(origin_i, target_i, sid):
            rows = pl.ds(origin_i * m_per, m_per)
            return pltpu.make_async_remote_copy(
                src_ref=xfull_ref.at[rows, :],
                dst_ref=xfull_ref.at[rows, :],
                send_sem=send_sems.at[sid],
                recv_sem=recv_sems.at[sid],
                device_id=(target_i,),
                device_id_type=pl.DeviceIdType.MESH,
            )

        barrier_sem = pltpu.get_barrier_semaphore()
        nbrs = []
        for dz in range(1, NZ):
            nbrs.append(idx_same_x(my_y, (my_z + dz) % NZ))
        for dy in range(1, NY):
            nbrs.append(idx_same_x((my_y + dy) % NY, my_z))
        nbrs.append(me ^ 1)
        for t in nbrs:
            pl.semaphore_signal(
                barrier_sem, inc=1,
                device_id=(t,), device_id_type=pl.DeviceIdType.MESH,
            )
        pl.semaphore_wait(barrier_sem, len(nbrs))

        xfull_ref[pl.ds(me * m_per, m_per), :] = x_ref[:, :].astype(jnp.bfloat16)

        sends = []

        def push(origin_i, target_i, sid):
            rdma = chunk_copy(origin_i, target_i, sid)
            rdma.start()
            sends.append(rdma)

        xpart = me ^ 1

        for dz in range(1, NZ):
            push(me, idx_same_x(my_y, (my_z + dz) % NZ), _ZID(dz))
        for dy in range(1, NY):
            push(me, idx_same_x((my_y + dy) % NY, my_z), _YID(dy, 0))
        push(me, xpart, _XID(0, 0))

        for dz in range(1, NZ):
            zo = (my_z + NZ - dz) % NZ
            origin = idx_same_x(my_y, zo)
            chunk_copy(origin, me, _ZID(dz)).wait_recv()
            s = (zo + NZ - my_z) % NZ
            for dy in range(1, NY):
                push(origin, idx_same_x((my_y + dy) % NY, my_z), _YID(dy, s))
            push(origin, xpart, _XID(0, s))

        for s in (0, 3, 2, 1):
            for dy in range(1, NY):
                yo = (my_y + NY - dy) % NY
                zo = (my_z + s) % NZ
                origin = idx_same_x(yo, zo)
                chunk_copy(origin, me, _YID(dy, s)).wait_recv()
                dyo = (yo + NY - my_y) % NY
                push(origin, xpart, _XID(dyo, s))

        xwaits = [(0, 0)] + [(0, s) for s in (3, 2, 1)]
        xwaits += [((NY - dy) % NY, s) for s in (0, 3, 2, 1) for dy in range(1, NY)]
        for dyo, dzo in xwaits:
            yo = (my_y + dyo) % NY
            zo = (my_z + dzo) % NZ
            origin = 8 * zo + 2 * yo + jnp.bitwise_xor(
                jnp.bitwise_xor(b, jnp.bitwise_and(my_y ^ yo, 1)), 1
            )
            chunk_copy(origin, me, _XID(dyo, dzo)).wait_recv()

        y = lax.dot_general(
            xfull_ref[:, :], w_ref[:, :].astype(jnp.bfloat16),
            (((1,), (0,)), ((), ())),
            preferred_element_type=jnp.float32,
        )
        c = 0.7978845608028654
        out_ref[:, :] = 0.5 * y * (1.0 + jnp.tanh(c * (y + 0.044715 * y * y * y)))

        for rdma in sends:
            rdma.wait_send()

    return pl.pallas_call(
        body,
        out_shape=jax.ShapeDtypeStruct((m, n_per), jnp.float32),
        in_specs=[
            pl.BlockSpec(memory_space=pltpu.VMEM),
            pl.BlockSpec(memory_space=pltpu.VMEM),
        ],
        out_specs=pl.BlockSpec(memory_space=pltpu.VMEM),
        scratch_shapes=[
            pltpu.VMEM((m, k), jnp.bfloat16),
            pltpu.SemaphoreType.DMA((N_SEMS,)),
            pltpu.SemaphoreType.DMA((N_SEMS,)),
        ],
        compiler_params=pltpu.CompilerParams(collective_id=0),
    )(x, w_mat)


# device time: 23209 ns/iter; 1.5193x vs baseline; 1.1344x over previous
import jax
import jax.numpy as jnp
from jax import lax
from jax.experimental import pallas as pl
from jax.experimental.pallas import tpu as pltpu

N_DEV = 32
NZ = 4
NY = 4

_AZ = lambda dz: dz - 1
_AY = lambda dy, s: 3 + (dy - 1) * 4 + s
_AX = lambda dyo, dzo: 15 + dyo * 4 + dzo
_BY = lambda dy: 31 + dy - 1
_BZ = lambda dz, s: 34 + (dz - 1) * 4 + s
_BX = lambda dyo, dzo: 46 + dyo * 4 + dzo
N_SEMS = 62


def kernel(x, w_mat):
    m_per, k = x.shape
    _, n_per = w_mat.shape
    m = N_DEV * m_per
    kh = k // 2

    def body(x_ref, w_ref, out_ref, xfull_ref, send_sems, recv_sems):
        me = lax.axis_index("i")
        my_z = me // 8
        r = me % 8
        my_y = r // 2
        b = r % 2

        def idx_same_x(y, z):
            return 8 * z + 2 * y + jnp.bitwise_xor(b, jnp.bitwise_and(my_y ^ y, 1))

        def idx_other_x(y, z):
            return jnp.bitwise_xor(idx_same_x(y, z), 1)

        def half_copy(origin_i, target_i, sid, c0):
            return pltpu.make_async_remote_copy(
                src_ref=xfull_ref.at[pl.ds(origin_i * m_per, m_per), pl.ds(c0, kh)],
                dst_ref=xfull_ref.at[pl.ds(origin_i * m_per, m_per), pl.ds(c0, kh)],
                send_sem=send_sems.at[sid],
                recv_sem=recv_sems.at[sid],
                device_id=(target_i,),
                device_id_type=pl.DeviceIdType.MESH,
            )

        barrier_sem = pltpu.get_barrier_semaphore()
        z_peers = [idx_same_x(my_y, (my_z + dz) % NZ) for dz in range(1, NZ)]
        y_peers = [idx_same_x((my_y + dy) % NY, my_z) for dy in range(1, NY)]
        xpart = me ^ 1
        for t in z_peers + y_peers + [xpart]:
            pl.semaphore_signal(
                barrier_sem, inc=1,
                device_id=(t,), device_id_type=pl.DeviceIdType.MESH,
            )
        pl.semaphore_wait(barrier_sem, 7)

        xfull_ref[pl.ds(me * m_per, m_per), :] = x_ref[:, :].astype(jnp.bfloat16)

        sends = []

        def push(origin_i, target_i, sid, c0):
            rdma = half_copy(origin_i, target_i, sid, c0)
            rdma.start()
            sends.append(rdma)

        for dz in range(1, NZ):
            push(me, z_peers[dz - 1], _AZ(dz), 0)
        for dy in range(1, NY):
            push(me, y_peers[dy - 1], _BY(dy), kh)
        for dy in range(1, NY):
            push(me, y_peers[dy - 1], _AY(dy, 0), 0)
        for dz in range(1, NZ):
            push(me, z_peers[dz - 1], _BZ(dz, 0), kh)
        push(me, xpart, _AX(0, 0), 0)
        push(me, xpart, _BX(0, 0), kh)

        for d in range(1, NZ):
            zo = (my_z + NZ - d) % NZ
            origin = idx_same_x(my_y, zo)
            half_copy(origin, me, _AZ(d), 0).wait_recv()
            s = (NZ - d) % NZ
            for dy in range(1, NY):
                push(origin, y_peers[dy - 1], _AY(dy, s), 0)
            push(origin, xpart, _AX(0, s), 0)
            yo = (my_y + NY - d) % NY
            origin = idx_same_x(yo, my_z)
            half_copy(origin, me, _BY(d), kh).wait_recv()
            s = (NY - d) % NY
            for dz in range(1, NZ):
                push(origin, z_peers[dz - 1], _BZ(dz, s), kh)
            push(origin, xpart, _BX(s, 0), kh)

        for s in (0, 3, 2, 1):
            for d in range(1, NY):
                yo = (my_y + NY - d) % NY
                zo = (my_z + s) % NZ
                origin = idx_same_x(yo, zo)
                rec = half_copy(origin, me, _AY(d, s), 0)
                rec.wait_recv()
                push(origin, xpart, _AX((NY - d) % NY, s), 0)
                yo = (my_y + s) % NY
                zo = (my_z + NZ - d) % NZ
                origin = idx_same_x(yo, zo)
                rec = half_copy(origin, me, _BZ(d, s), kh)
                rec.wait_recv()
                push(origin, xpart, _BX(s, (NZ - d) % NZ), kh)

        xwaits = [(_AX, 0, 0, 0), (_BX, 0, 0, kh)]
        for d in (1, 2, 3):
            xwaits.append((_AX, 0, (NZ - d) % NZ, 0))
            xwaits.append((_BX, (NY - d) % NY, 0, kh))
        for s in (0, 3, 2, 1):
            for d in (1, 2, 3):
                xwaits.append((_AX, (NY - d) % NY, s, 0))
                xwaits.append((_BX, s, (NZ - d) % NZ, kh))
        for fid, dyo, dzo, c0 in xwaits:
            yo = (my_y + dyo) % NY
            zo = (my_z + dzo) % NZ
            half_copy(idx_other_x(yo, zo), me, fid(dyo, dzo), c0).wait_recv()

        y = lax.dot_general(
            xfull_ref[:, :], w_ref[:, :].astype(jnp.bfloat16),
            (((1,), (0,)), ((), ())),
            preferred_element_type=jnp.float32,
        )
        c = 0.7978845608028654
        out_ref[:, :] = 0.5 * y * (1.0 + jnp.tanh(c * (y + 0.044715 * y * y * y)))

        for rdma in sends:
            rdma.wait_send()

    return pl.pallas_call(
        body,
        out_shape=jax.ShapeDtypeStruct((m, n_per), jnp.float32),
        in_specs=[
            pl.BlockSpec(memory_space=pltpu.VMEM),
            pl.BlockSpec(memory_space=pltpu.VMEM),
        ],
        out_specs=pl.BlockSpec(memory_space=pltpu.VMEM),
        scratch_shapes=[
            pltpu.VMEM((m, k), jnp.bfloat16),
            pltpu.SemaphoreType.DMA((N_SEMS,)),
            pltpu.SemaphoreType.DMA((N_SEMS,)),
        ],
        compiler_params=pltpu.CompilerParams(collective_id=0),
    )(x, w_mat)
